# Initial kernel scaffold; baseline (speedup 1.0000x reference)
import jax
import jax.numpy as jnp
from jax.experimental import pallas as pl


def kernel(boxes, scores, feats, W1, b1, W2, b2):
    raise NotImplementedError("write your pallas kernel here")


# speedup vs baseline: 27.5408x; 27.5408x over previous
import jax
import jax.numpy as jnp
from jax.experimental import pallas as pl
from jax.experimental.pallas import tpu as pltpu

N = 5000
NP = 5120
B = 128
NB = NP // B
TOPK = 300
TK = 384
TH = 0.05
FD = 128

_f32 = jnp.float32


def _nms_body(cols, rows, feats, W1, b1, W2, b2, out_s, out_c,
              scols, srows, rankc, rankr, keepr, suppr, r2row, r2org):
    s_row = rows[4:5, :]
    s_col = cols[:, 4:5]
    j_lane = jax.lax.broadcasted_iota(jnp.int32, (1, NP), 1)
    j_sub = jax.lax.broadcasted_iota(jnp.int32, (NP, 1), 0)

    def rank_blk(bi, carry):
        sb = cols[pl.ds(bi * B, B), 4:5]
        ib = jax.lax.broadcasted_iota(jnp.int32, (B, 1), 0) + bi * B
        hit = (s_row > sb) | ((s_row == sb) & (j_lane < ib))
        rankc[pl.ds(bi * B, B), :] = jnp.sum(hit.astype(_f32), axis=1,
                                             keepdims=True)
        sbr = rows[4:5, pl.ds(bi * B, B)]
        ibr = jax.lax.broadcasted_iota(jnp.int32, (1, B), 1) + bi * B
        hit2 = (s_col > sbr) | ((s_col == sbr) & (j_sub < ibr))
        rankr[0:1, pl.ds(bi * B, B)] = jnp.sum(hit2.astype(_f32), axis=0,
                                               keepdims=True)
        return carry

    jax.lax.fori_loop(0, NB, rank_blk, 0)

    def sort_blk(bi, carry):
        t_sub = (jax.lax.broadcasted_iota(jnp.int32, (B, 1), 0)
                 + bi * B).astype(_f32)
        P = (rankr[0:1, :] == t_sub).astype(_f32)
        scols[pl.ds(bi * B, B), :] = jnp.dot(
            P, cols[...], preferred_element_type=_f32,
            precision=jax.lax.Precision.HIGHEST)
        t_lane = (jax.lax.broadcasted_iota(jnp.int32, (1, B), 1)
                  + bi * B).astype(_f32)
        PT = (rankc[:, 0:1] == t_lane).astype(_f32)
        srows[:, pl.ds(bi * B, B)] = jnp.dot(
            rows[...], PT, preferred_element_type=_f32,
            precision=jax.lax.Precision.HIGHEST)
        return carry

    jax.lax.fori_loop(0, NB, sort_blk, 0)

    suppr[...] = jnp.zeros((1, NP), _f32)
    tri = (jax.lax.broadcasted_iota(jnp.int32, (B, B), 0)
           < jax.lax.broadcasted_iota(jnp.int32, (B, B), 1)).astype(_f32)

    def nms_blk(bi, carry):
        zc = scols[pl.ds(bi * B, B), 0:1]
        yc = scols[pl.ds(bi * B, B), 1:2]
        xc = scols[pl.ds(bi * B, B), 2:3]
        dc = scols[pl.ds(bi * B, B), 3:4]
        z_r = srows[0:1, :]
        y_r = srows[1:2, :]
        x_r = srows[2:3, :]
        d_r = srows[3:4, :]
        rc = dc / 2.0
        rr = d_r / 2.0
        ovz = jnp.clip(jnp.minimum(zc + rc, z_r + rr)
                       - jnp.maximum(zc - rc, z_r - rr), 0.0, None)
        ovy = jnp.clip(jnp.minimum(yc + rc, y_r + rr)
                       - jnp.maximum(yc - rc, y_r - rr), 0.0, None)
        ovx = jnp.clip(jnp.minimum(xc + rc, x_r + rr)
                       - jnp.maximum(xc - rc, x_r - rr), 0.0, None)
        inter = ovz * ovy * ovx
        union = dc * dc * dc + d_r * d_r * d_r - inter
        m = ((inter / union) >= TH).astype(_f32)

        zb = srows[0:1, pl.ds(bi * B, B)]
        yb = srows[1:2, pl.ds(bi * B, B)]
        xb = srows[2:3, pl.ds(bi * B, B)]
        db = srows[3:4, pl.ds(bi * B, B)]
        rb_ = db / 2.0
        bvz = jnp.clip(jnp.minimum(zc + rc, zb + rb_)
                       - jnp.maximum(zc - rc, zb - rb_), 0.0, None)
        bvy = jnp.clip(jnp.minimum(yc + rc, yb + rb_)
                       - jnp.maximum(yc - rc, yb - rb_), 0.0, None)
        bvx = jnp.clip(jnp.minimum(xc + rc, xb + rb_)
                       - jnp.maximum(xc - rc, xb - rb_), 0.0, None)
        binter = bvz * bvy * bvx
        bunion = dc * dc * dc + db * db * db - binter
        sub = ((binter / bunion) >= TH).astype(_f32) * tri
        init = 1.0 - suppr[0:1, pl.ds(bi * B, B)]

        def w_cond(st):
            k, prev, it = st
            return jnp.logical_and(jnp.any(k != prev), it < B)

        def w_step(st):
            k, prev, it = st
            hits = jnp.dot(k, sub, preferred_element_type=_f32)
            knew = init * (hits == 0.0).astype(_f32)
            return (knew, k, it + 1)

        kfin, _, _ = jax.lax.while_loop(
            w_cond, w_step, (init, init - 1.0, jnp.int32(0)))
        keepr[0:1, pl.ds(bi * B, B)] = kfin

        contrib = jnp.dot(kfin, m, preferred_element_type=_f32)
        later = (j_lane >= (bi + 1) * B).astype(_f32)
        suppr[...] = jnp.maximum(suppr[...],
                                 (contrib > 0.0).astype(_f32) * later)
        return carry

    jax.lax.fori_loop(0, NB, nms_blk, 0)

    keep = keepr[...]
    out_s[...] = srows[4:5, :] * keep

    real = (j_lane < N).astype(_f32)
    keepR = keep * real
    Ktot = jnp.sum(keepR)
    U = tri

    def pf_blk(bi, carry):
        ck, cn = carry
        posb = jax.lax.broadcasted_iota(jnp.int32, (1, B), 1) + bi * B
        rb = (posb < N).astype(_f32)
        kb = keepr[0:1, pl.ds(bi * B, B)]
        vk = kb * rb
        vn = rb * (1.0 - kb)
        A = jnp.dot(vk, U, preferred_element_type=_f32) + ck
        C = jnp.dot(vn, U, preferred_element_type=_f32) + cn
        r2 = vk * A + vn * (Ktot + C) + (1.0 - rb) * 1e9
        r2row[0:1, pl.ds(bi * B, B)] = r2
        return (ck + jnp.sum(vk), cn + jnp.sum(vn))

    jax.lax.fori_loop(0, NB, pf_blk, (_f32(0.0), _f32(0.0)))

    p_sub = jax.lax.broadcasted_iota(jnp.int32, (NP, 1), 0).astype(_f32)

    def map_blk(bi, carry):
        rk = rankr[0:1, pl.ds(bi * B, B)]
        PT2 = (p_sub == rk).astype(_f32)
        r2org[0:1, pl.ds(bi * B, B)] = jnp.dot(
            r2row[...], PT2, preferred_element_type=_f32,
            precision=jax.lax.Precision.HIGHEST)
        return carry

    jax.lax.fori_loop(0, NB, map_blk, 0)

    slot = jax.lax.broadcasted_iota(jnp.int32, (TK, 1), 0).astype(_f32)
    Gsel = ((slot == r2org[...]) & (slot < float(TOPK))).astype(_f32)
    fsel = jnp.dot(Gsel, feats[...], preferred_element_type=_f32,
                   precision=jax.lax.Precision.HIGHEST)
    h = jnp.maximum(jnp.dot(fsel, W1[...], preferred_element_type=_f32,
                            precision=jax.lax.Precision.HIGHEST)
                    + b1[...], 0.0)
    out_c[...] = jnp.dot(h, W2[...], preferred_element_type=_f32,
                         precision=jax.lax.Precision.HIGHEST) + b2[...]


def kernel(boxes, scores, feats, W1, b1, W2, b2):
    npad = NP - N
    pad_c = 1.0e6 + jnp.arange(npad, dtype=_f32)[:, None] * 1000.0
    pad_boxes = jnp.concatenate(
        [jnp.broadcast_to(pad_c, (npad, 3)),
         jnp.full((npad, 1), 10.0, _f32)], axis=1)
    boxes_p = jnp.concatenate([boxes, pad_boxes], axis=0)
    s_p = jnp.concatenate([scores, jnp.full((npad,), -1e30, _f32)])
    cols = jnp.concatenate(
        [boxes_p, s_p[:, None], jnp.zeros((NP, 3), _f32)], axis=1)
    rows = cols.T
    feats_p = jnp.concatenate([feats, jnp.zeros((npad, FD), _f32)], axis=0)
    W2p = jnp.concatenate([W2, jnp.zeros((64, 127), _f32)], axis=1)
    b1r = b1[None, :]
    b2r = jnp.broadcast_to(b2[0], (1, 128)).astype(_f32)

    out_s, out_c = pl.pallas_call(
        _nms_body,
        out_shape=[jax.ShapeDtypeStruct((1, NP), _f32),
                   jax.ShapeDtypeStruct((TK, 128), _f32)],
        scratch_shapes=[
            pltpu.VMEM((NP, 8), _f32),
            pltpu.VMEM((8, NP), _f32),
            pltpu.VMEM((NP, 1), _f32),
            pltpu.VMEM((1, NP), _f32),
            pltpu.VMEM((1, NP), _f32),
            pltpu.VMEM((1, NP), _f32),
            pltpu.VMEM((1, NP), _f32),
            pltpu.VMEM((1, NP), _f32),
        ],
    )(cols, rows, feats_p, W1, b1r, W2p, b2r)

    return jnp.concatenate([out_s[0, :N], out_c[:TOPK, 0]])
